# baseline (speedup 1.0000x reference)
import functools

import jax
import jax.numpy as jnp
from jax import lax
from jax.experimental import pallas as pl
from jax.experimental.pallas import tpu as pltpu
from jax.experimental.pallas import tpu_sc as plsc

_NUM_BUCKETS = 32
_NUM_HEADS = 16
_Q_LEN = 2048
_K_LEN = 2048

_QT = 8
_PATCH_W = 256
_NPATCH = 32
_LB_W = 512
_FQT = 16
_TILES_PER_WORKER = _Q_LEN // 2 // _QT
_FILLS_PER_WORKER = _Q_LEN // 2 // _FQT
_ROUND = 32

_THRESHOLDS = (19, 21, 24, 27, 31, 35, 40, 46, 52, 59, 67, 77, 87, 99, 113)


def _sc_body(table_hbm, out_hbm, tab_v, lb_v, const_v, patch_v, fsa, fsb, psem):
    head = lax.axis_index("s")
    half = lax.axis_index("c")
    pltpu.sync_copy(table_hbm, tab_v)

    lane = lax.broadcasted_iota(jnp.int32, (16,), 0)
    h16 = jnp.full((16,), head, jnp.int32)
    v31 = plsc.load_gather(
        tab_v, [jnp.full((16,), _NUM_BUCKETS - 1, jnp.int32), h16]
    )

    def fill_chunk(t, _):
        const_v[t % _FQT, pl.ds((t // _FQT) * 16, 16)] = v31
        return 0

    lax.fori_loop(0, _FQT * _K_LEN // 16, fill_chunk, 0)

    fills, patches = [], []

    for tf in range(_FILLS_PER_WORKER):
        q0f = half * (_Q_LEN // 2) + tf * _FQT
        q0f = pl.multiple_of(q0f, _FQT)
        fills.append(pltpu.make_async_copy(
            const_v,
            out_hbm.at[head, pl.ds(q0f, _FQT), :],
            fsa if (tf // _ROUND) % 2 == 0 else fsb,
        ))
    for t in range(_TILES_PER_WORKER):
        q0 = half * (_Q_LEN // 2) + t * _QT
        a = jnp.clip((q0 - 112) & -128, 0, _K_LEN - _PATCH_W)
        si = (q0 - a) >> 3
        q0 = pl.multiple_of(q0, _QT)
        a = pl.multiple_of(a, 128)
        patches.append(pltpu.make_async_copy(
            patch_v.at[si],
            out_hbm.at[head, pl.ds(q0, _QT), pl.ds(a, _PATCH_W)],
            psem,
        ))

    for j in range(_ROUND):
        fills[j].start()

    def line_chunk(t, _):
        d = t * 16 + lane - (_LB_W // 2 - 1)
        n = jnp.abs(d)
        large = jnp.full((16,), _NUM_BUCKETS // 2, jnp.int32)
        for tj in _THRESHOLDS:
            large = large + (n >= tj).astype(jnp.int32)
        neg_b = jnp.where(n < _NUM_BUCKETS // 2, n, large)
        pos_b = _NUM_BUCKETS // 2 + jnp.minimum(d, _NUM_BUCKETS // 2 - 1)
        b = jnp.where(d > 0, pos_b, neg_b)
        lb_v[t, :] = plsc.load_gather(tab_v, [b, h16])
        return 0

    lax.fori_loop(0, _LB_W // 16, line_chunk, 0)

    def patch_row(t, _):
        si = t // _QT
        r = t % _QT
        j0 = (_LB_W // 2 - 1) - 8 * si - r + lane
        for m in range(_PATCH_W // 16):
            j = j0 + 16 * m
            patch_v[si, r, pl.ds(m * 16, 16)] = plsc.load_gather(
                lb_v, [j >> 4, j & 15]
            )
        return 0

    lax.fori_loop(0, _NPATCH * _QT, patch_row, 0)

    nrounds = _FILLS_PER_WORKER // _ROUND
    ppr = _ROUND * _FQT // _QT
    for k in range(nrounds):
        if k + 1 < nrounds:
            for j in range(_ROUND):
                fills[(k + 1) * _ROUND + j].start()
        for j in range(_ROUND):
            fills[k * _ROUND + j].wait()
        for j in range(ppr):
            patches[k * ppr + j].start()
    for p in patches:
        p.wait()


def _expand(table):
    mesh = plsc.VectorSubcoreMesh(core_axis_name="c", subcore_axis_name="s")
    run = functools.partial(
        pl.kernel,
        mesh=mesh,
        out_type=jax.ShapeDtypeStruct((_NUM_HEADS, _Q_LEN, _K_LEN), jnp.float32),
        scratch_types=[
            pltpu.VMEM((_NUM_BUCKETS, _NUM_HEADS), jnp.float32),
            pltpu.VMEM((_LB_W // 16, 16), jnp.float32),
            pltpu.VMEM((_FQT, _K_LEN), jnp.float32),
            pltpu.VMEM((_NPATCH, _QT, _PATCH_W), jnp.float32),
            pltpu.SemaphoreType.DMA,
            pltpu.SemaphoreType.DMA,
            pltpu.SemaphoreType.DMA,
        ],
        compiler_params=pltpu.CompilerParams(
            use_tc_tiling_on_sc=True, needs_layout_passes=False
        ),
    )(_sc_body)
    return run(table)


def kernel(q_len, k_len, table):
    del q_len, k_len
    return _expand(table)


# speedup vs baseline: 1.0281x; 1.0281x over previous
import functools

import jax
import jax.numpy as jnp
from jax import lax
from jax.experimental import pallas as pl
from jax.experimental.pallas import tpu as pltpu
from jax.experimental.pallas import tpu_sc as plsc

_NUM_BUCKETS = 32
_NUM_HEADS = 16
_Q_LEN = 2048
_K_LEN = 2048

_QT = 8
_PATCH_W = 256
_NPATCH = 32
_LB_W = 512
_FQT = 8
_TILES_PER_WORKER = _Q_LEN // 2 // _QT
_FILLS_PER_WORKER = _Q_LEN // 2 // _FQT
_ROUNDS = (64, 32, 16, 8, 8)

_THRESHOLDS = (19, 21, 24, 27, 31, 35, 40, 46, 52, 59, 67, 77, 87, 99, 113)


def _sc_body(table_hbm, out_hbm, tab_v, lb_v, const_v, patch_v, fsa, fsb, psem):
    head = lax.axis_index("s")
    half = lax.axis_index("c")
    pltpu.sync_copy(table_hbm, tab_v)

    lane = lax.broadcasted_iota(jnp.int32, (16,), 0)
    h16 = jnp.full((16,), head, jnp.int32)
    v31 = plsc.load_gather(
        tab_v, [jnp.full((16,), _NUM_BUCKETS - 1, jnp.int32), h16]
    )

    def fill_chunk(t, _):
        const_v[t % _FQT, pl.ds((t // _FQT) * 16, 16)] = v31
        return 0

    lax.fori_loop(0, _FQT * _K_LEN // 16, fill_chunk, 0)

    fills, patches = [], []

    bounds = [0]
    for r in _ROUNDS:
        bounds.append(bounds[-1] + r)
    assert bounds[-1] == _FILLS_PER_WORKER

    def round_of(tf):
        for k in range(len(_ROUNDS)):
            if tf < bounds[k + 1]:
                return k
        raise AssertionError

    for tf in range(_FILLS_PER_WORKER):
        q0f = half * (_Q_LEN // 2) + tf * _FQT
        q0f = pl.multiple_of(q0f, _FQT)
        fills.append(pltpu.make_async_copy(
            const_v,
            out_hbm.at[head, pl.ds(q0f, _FQT), :],
            fsa if round_of(tf) % 2 == 0 else fsb,
        ))
    for t in range(_TILES_PER_WORKER):
        q0 = half * (_Q_LEN // 2) + t * _QT
        a = jnp.clip((q0 - 112) & -128, 0, _K_LEN - _PATCH_W)
        si = (q0 - a) >> 3
        q0 = pl.multiple_of(q0, _QT)
        a = pl.multiple_of(a, 128)
        patches.append(pltpu.make_async_copy(
            patch_v.at[si],
            out_hbm.at[head, pl.ds(q0, _QT), pl.ds(a, _PATCH_W)],
            psem,
        ))

    for j in range(bounds[1]):
        fills[j].start()

    def line_chunk(t, _):
        d = t * 16 + lane - (_LB_W // 2 - 1)
        n = jnp.abs(d)
        large = jnp.full((16,), _NUM_BUCKETS // 2, jnp.int32)
        for tj in _THRESHOLDS:
            large = large + (n >= tj).astype(jnp.int32)
        neg_b = jnp.where(n < _NUM_BUCKETS // 2, n, large)
        pos_b = _NUM_BUCKETS // 2 + jnp.minimum(d, _NUM_BUCKETS // 2 - 1)
        b = jnp.where(d > 0, pos_b, neg_b)
        lb_v[t, :] = plsc.load_gather(tab_v, [b, h16])
        return 0

    lax.fori_loop(0, _LB_W // 16, line_chunk, 0)

    def patch_row(t, _):
        si = t // _QT
        r = t % _QT
        j0 = (_LB_W // 2 - 1) - 8 * si - r + lane
        for m in range(_PATCH_W // 16):
            j = j0 + 16 * m
            patch_v[si, r, pl.ds(m * 16, 16)] = plsc.load_gather(
                lb_v, [j >> 4, j & 15]
            )
        return 0

    lax.fori_loop(0, _NPATCH * _QT, patch_row, 0)

    ppf = _FQT // _QT
    for k in range(len(_ROUNDS)):
        if k + 1 < len(_ROUNDS):
            for tf in range(bounds[k + 1], bounds[k + 2]):
                fills[tf].start()
        for tf in range(bounds[k], bounds[k + 1]):
            fills[tf].wait()
        for t in range(bounds[k] * ppf, bounds[k + 1] * ppf):
            patches[t].start()
    for p in patches:
        p.wait()


def _expand(table):
    mesh = plsc.VectorSubcoreMesh(core_axis_name="c", subcore_axis_name="s")
    run = functools.partial(
        pl.kernel,
        mesh=mesh,
        out_type=jax.ShapeDtypeStruct((_NUM_HEADS, _Q_LEN, _K_LEN), jnp.float32),
        scratch_types=[
            pltpu.VMEM((_NUM_BUCKETS, _NUM_HEADS), jnp.float32),
            pltpu.VMEM((_LB_W // 16, 16), jnp.float32),
            pltpu.VMEM((_FQT, _K_LEN), jnp.float32),
            pltpu.VMEM((_NPATCH, _QT, _PATCH_W), jnp.float32),
            pltpu.SemaphoreType.DMA,
            pltpu.SemaphoreType.DMA,
            pltpu.SemaphoreType.DMA,
        ],
        compiler_params=pltpu.CompilerParams(
            use_tc_tiling_on_sc=True, needs_layout_passes=False
        ),
    )(_sc_body)
    return run(table)


def kernel(q_len, k_len, table):
    del q_len, k_len
    return _expand(table)


# speedup vs baseline: 1.0290x; 1.0008x over previous
import functools

import jax
import jax.numpy as jnp
from jax import lax
from jax.experimental import pallas as pl
from jax.experimental.pallas import tpu as pltpu
from jax.experimental.pallas import tpu_sc as plsc

_NUM_BUCKETS = 32
_NUM_HEADS = 16
_Q_LEN = 2048
_K_LEN = 2048

_QT = 8
_PATCH_W = 256
_NPATCH = 32
_LB_W = 512
_FQT = 8
_TILES_PER_WORKER = _Q_LEN // 2 // _QT
_FILLS_PER_WORKER = _Q_LEN // 2 // _FQT
_ROUND = 32
_NROUNDS = _FILLS_PER_WORKER // _ROUND

_THRESHOLDS = (19, 21, 24, 27, 31, 35, 40, 46, 52, 59, 67, 77, 87, 99, 113)


def _sc_body(table_hbm, out_hbm, tab_v, lb_v, const_v, patch_v, fsa, fsb, psem):
    head = lax.axis_index("s")
    half = lax.axis_index("c")
    pltpu.sync_copy(table_hbm, tab_v)

    lane = lax.broadcasted_iota(jnp.int32, (16,), 0)
    h16 = jnp.full((16,), head, jnp.int32)
    v31 = plsc.load_gather(
        tab_v, [jnp.full((16,), _NUM_BUCKETS - 1, jnp.int32), h16]
    )

    def fill_chunk(t, _):
        const_v[t % _FQT, pl.ds((t // _FQT) * 16, 16)] = v31
        return 0

    lax.fori_loop(0, _FQT * _K_LEN // 16, fill_chunk, 0)

    fills, patches = [], []

    for tf in range(_FILLS_PER_WORKER):
        q0f = half * (_Q_LEN // 2) + tf * _FQT
        q0f = pl.multiple_of(q0f, _FQT)
        fills.append(pltpu.make_async_copy(
            const_v,
            out_hbm.at[head, pl.ds(q0f, _FQT), :],
            fsa if (tf // _ROUND) % 2 == 0 else fsb,
        ))
    for t in range(_TILES_PER_WORKER):
        q0 = half * (_Q_LEN // 2) + t * _QT
        a = jnp.clip((q0 - 112) & -128, 0, _K_LEN - _PATCH_W)
        si = (q0 - a) >> 3
        q0 = pl.multiple_of(q0, _QT)
        a = pl.multiple_of(a, 128)
        patches.append(pltpu.make_async_copy(
            patch_v.at[si],
            out_hbm.at[head, pl.ds(q0, _QT), pl.ds(a, _PATCH_W)],
            psem,
        ))

    for j in range(_ROUND):
        fills[j].start()

    def line_chunk(t, _):
        d = t * 16 + lane - (_LB_W // 2 - 1)
        n = jnp.abs(d)
        large = jnp.full((16,), _NUM_BUCKETS // 2, jnp.int32)
        for tj in _THRESHOLDS:
            large = large + (n >= tj).astype(jnp.int32)
        neg_b = jnp.where(n < _NUM_BUCKETS // 2, n, large)
        pos_b = _NUM_BUCKETS // 2 + jnp.minimum(d, _NUM_BUCKETS // 2 - 1)
        b = jnp.where(d > 0, pos_b, neg_b)
        lb_v[t, :] = plsc.load_gather(tab_v, [b, h16])
        return 0

    lax.fori_loop(0, _LB_W // 16, line_chunk, 0)

    def patch_row(t, _):
        si = t // _QT
        r = t % _QT
        j0 = (_LB_W // 2 - 1) - 8 * si - r + lane
        for m in range(_PATCH_W // 16):
            j = j0 + 16 * m
            patch_v[si, r, pl.ds(m * 16, 16)] = plsc.load_gather(
                lb_v, [j >> 4, j & 15]
            )
        return 0

    lax.fori_loop(0, _NPATCH * _QT, patch_row, 0)

    for k in range(_NROUNDS):
        if k + 1 < _NROUNDS:
            for j in range(_ROUND):
                fills[(k + 1) * _ROUND + j].start()
        for j in range(_ROUND):
            fills[k * _ROUND + j].wait()
    ppr = _ROUND * _FQT // _QT
    for k in range(_NROUNDS):
        for t in range(k * ppr, (k + 1) * ppr):
            patches[t].start()
        for t in range(k * ppr, (k + 1) * ppr):
            patches[t].wait()


def _expand(table):
    mesh = plsc.VectorSubcoreMesh(core_axis_name="c", subcore_axis_name="s")
    run = functools.partial(
        pl.kernel,
        mesh=mesh,
        out_type=jax.ShapeDtypeStruct((_NUM_HEADS, _Q_LEN, _K_LEN), jnp.float32),
        scratch_types=[
            pltpu.VMEM((_NUM_BUCKETS, _NUM_HEADS), jnp.float32),
            pltpu.VMEM((_LB_W // 16, 16), jnp.float32),
            pltpu.VMEM((_FQT, _K_LEN), jnp.float32),
            pltpu.VMEM((_NPATCH, _QT, _PATCH_W), jnp.float32),
            pltpu.SemaphoreType.DMA,
            pltpu.SemaphoreType.DMA,
            pltpu.SemaphoreType.DMA,
        ],
        compiler_params=pltpu.CompilerParams(
            use_tc_tiling_on_sc=True, needs_layout_passes=False
        ),
    )(_sc_body)
    return run(table)


def kernel(q_len, k_len, table):
    del q_len, k_len
    return _expand(table)
